# Initial kernel scaffold; baseline (speedup 1.0000x reference)
import jax
import jax.numpy as jnp
from jax.experimental import pallas as pl


def kernel(x, edge_index, q, W1, b1, W2, b2, Ws1, bs1, Ws2, bs2):
    raise NotImplementedError("write your pallas kernel here")


# speedup vs baseline: 3.9297x; 3.9297x over previous
import functools

import jax
import jax.numpy as jnp
from jax import lax
from jax.experimental import pallas as pl
from jax.experimental.pallas import tpu as pltpu
from jax.experimental.pallas import tpu_sc as plsc

N = 10000
E = 320000
D = 128
H = 256
QD = 128

NSUB = 16
NCORE = 2
BE = 128
EPAD = 327680
EPS = EPAD // NSUB
NBATCH = EPS // BE
NP = 10112
RPT = 624
ZPT = NP // NSUB

BLK = 2000


@functools.cache
def _make_sc_agg(col_split):
    F = 128
    mesh = plsc.VectorSubcoreMesh(core_axis_name="c", subcore_axis_name="s",
                                  num_cores=NCORE, num_subcores=NSUB)

    def body(src_hbm, dst_hbm, h_hbm, zf_hbm, sum_out,
             src_v, dst_v, rows_v, acc, sem):
        c = lax.axis_index("c")
        s = lax.axis_index("s")

        zbase = s * ZPT
        pltpu.sync_copy(zf_hbm, acc.at[pl.ds(zbase, ZPT)])
        plsc.subcore_barrier()

        if col_split:
            ebase = s * EPS
            nbatch = EPS // BE
        else:
            wid = s * NCORE + c
            ebase = wid * (EPAD // (NSUB * NCORE))
            nbatch = EPAD // (NSUB * NCORE) // BE

        def batch(i, carry):
            eoff = ebase + i * BE
            if col_split:
                pltpu.sync_copy(src_hbm.at[c, 0, pl.ds(eoff, BE)], src_v)
            else:
                pltpu.sync_copy(src_hbm.at[pl.ds(eoff, BE)], src_v)
            pltpu.sync_copy(dst_hbm.at[pl.ds(eoff, BE)], dst_v)
            pltpu.async_copy(h_hbm.at[src_v], rows_v, sem).wait()
            pltpu.sync_copy(rows_v, acc.at[dst_v], add=True)
            return carry
        lax.fori_loop(0, nbatch, batch, 0)
        plsc.subcore_barrier()

        pltpu.sync_copy(acc.at[pl.ds(zbase, ZPT)],
                        sum_out.at[c, pl.ds(zbase, ZPT)])

    return pl.kernel(
        body,
        out_type=jax.ShapeDtypeStruct((NCORE, NP, F), jnp.float32),
        mesh=mesh,
        scratch_types=[
            pltpu.VMEM((BE,), jnp.int32),
            pltpu.VMEM((BE,), jnp.int32),
            pltpu.VMEM((BE, F), jnp.float32),
            pltpu.VMEM_SHARED((NP, F), jnp.float32),
            pltpu.SemaphoreType.DMA,
        ])


@functools.cache
def _make_sc_cnt():
    mesh = plsc.VectorSubcoreMesh(core_axis_name="c", subcore_axis_name="s",
                                  num_cores=NCORE, num_subcores=NSUB)

    def body(dst_hbm, zf_hbm, ones_hbm, cnt_out, dst_v, ones_v, acc, sem):
        c = lax.axis_index("c")
        s = lax.axis_index("s")
        zbase = s * ZPT
        pltpu.sync_copy(zf_hbm, acc.at[pl.ds(zbase, ZPT)])
        pltpu.sync_copy(ones_hbm, ones_v)
        plsc.subcore_barrier()

        wid = s * NCORE + c
        ebase = wid * (EPAD // (NSUB * NCORE))
        nbatch = EPAD // (NSUB * NCORE) // BE

        def batch(i, carry):
            eoff = ebase + i * BE
            pltpu.sync_copy(dst_hbm.at[pl.ds(eoff, BE)], dst_v)
            pltpu.sync_copy(ones_v, acc.at[dst_v], add=True)
            return carry
        lax.fori_loop(0, nbatch, batch, 0)
        plsc.subcore_barrier()

        pltpu.sync_copy(acc.at[pl.ds(zbase, ZPT)],
                        cnt_out.at[c, pl.ds(zbase, ZPT)])

    return pl.kernel(
        body,
        out_type=jax.ShapeDtypeStruct((NCORE, NP, 128), jnp.float32),
        mesh=mesh,
        scratch_types=[
            pltpu.VMEM((BE,), jnp.int32),
            pltpu.VMEM((BE, 128), jnp.float32),
            pltpu.VMEM_SHARED((NP, 128), jnp.float32),
            pltpu.SemaphoreType.DMA,
        ])


def _l1_body(x_r, s1_r, cnt_r, q_r, W_r, b_r, h1_r, rc_r):
    cnt = cnt_r[0, :, 0:1] + cnt_r[1, :, 0:1] + 1.0
    rc = 1.0 / cnt
    x = x_r[...]
    agg = (s1_r[0] + s1_r[1] + x) * rc
    W = W_r[...]
    acc = jnp.dot(x, W[0:128], preferred_element_type=jnp.float32)
    acc = acc + jnp.dot(agg, W[128:256], preferred_element_type=jnp.float32)
    qc = jnp.dot(q_r[...], W[256:384], preferred_element_type=jnp.float32)
    h = jnp.maximum(acc + qc + b_r[...], 0.0)
    h1_r[0] = h[:, :128]
    h1_r[1] = h[:, 128:]
    rc_r[...] = rc


def _tc_layer1(x, s1, cntT, q2, W1, b1):
    return pl.pallas_call(
        _l1_body,
        grid=(N // BLK,),
        in_specs=[
            pl.BlockSpec((BLK, D), lambda i: (i, 0)),
            pl.BlockSpec((2, BLK, 128), lambda i: (0, i, 0)),
            pl.BlockSpec((2, BLK, 128), lambda i: (0, i, 0)),
            pl.BlockSpec((1, QD), lambda i: (0, 0)),
            pl.BlockSpec((2 * D + QD, H), lambda i: (0, 0)),
            pl.BlockSpec((1, H), lambda i: (0, 0)),
        ],
        out_specs=[
            pl.BlockSpec((2, BLK, H // 2), lambda i: (0, i, 0)),
            pl.BlockSpec((BLK, 1), lambda i: (i, 0)),
        ],
        out_shape=[
            jax.ShapeDtypeStruct((2, N, H // 2), jnp.float32),
            jax.ShapeDtypeStruct((N, 1), jnp.float32),
        ],
    )(x, s1, cntT, q2, W1, b1)


def _l2_body(h1_r, s2_r, rc_r, q_r, W2_r, b2_r, Ws1_r, bs1_r, Ws2_r, bs2_r,
             out_r):
    rc = rc_r[...]
    h1a = h1_r[0]
    h1b = h1_r[1]
    aggA = (s2_r[0] + h1a) * rc
    aggB = (s2_r[1] + h1b) * rc
    W2 = W2_r[...]
    acc = jnp.dot(h1a, W2[0:128], preferred_element_type=jnp.float32)
    acc = acc + jnp.dot(h1b, W2[128:256], preferred_element_type=jnp.float32)
    acc = acc + jnp.dot(aggA, W2[256:384], preferred_element_type=jnp.float32)
    acc = acc + jnp.dot(aggB, W2[384:512], preferred_element_type=jnp.float32)
    qc = jnp.dot(q_r[...], W2[512:640], preferred_element_type=jnp.float32)
    h2 = jnp.maximum(acc + qc + b2_r[...], 0.0)
    Ws1 = Ws1_r[...]
    sacc = jnp.dot(h2, Ws1[0:256], preferred_element_type=jnp.float32)
    sq = jnp.dot(q_r[...], Ws1[256:384], preferred_element_type=jnp.float32)
    sv = jnp.maximum(sacc + sq + bs1_r[...], 0.0)
    out_r[...] = (jnp.dot(sv, Ws2_r[...], preferred_element_type=jnp.float32)
                  + bs2_r[...])


def _tc_layer2(h1, s2, rc, q2, W2, b2, Ws1, bs1, Ws2, bs2):
    return pl.pallas_call(
        _l2_body,
        grid=(N // BLK,),
        in_specs=[
            pl.BlockSpec((2, BLK, H // 2), lambda i: (0, i, 0)),
            pl.BlockSpec((2, BLK, H // 2), lambda i: (0, i, 0)),
            pl.BlockSpec((BLK, 1), lambda i: (i, 0)),
            pl.BlockSpec((1, QD), lambda i: (0, 0)),
            pl.BlockSpec((2 * H + QD, H), lambda i: (0, 0)),
            pl.BlockSpec((1, H), lambda i: (0, 0)),
            pl.BlockSpec((H + QD, 128), lambda i: (0, 0)),
            pl.BlockSpec((1, 128), lambda i: (0, 0)),
            pl.BlockSpec((128, 1), lambda i: (0, 0)),
            pl.BlockSpec((1, 1), lambda i: (0, 0)),
        ],
        out_specs=pl.BlockSpec((BLK, 1), lambda i: (i, 0)),
        out_shape=jax.ShapeDtypeStruct((N, 1), jnp.float32),
    )(h1, s2, rc, q2, W2, b2, Ws1, bs1, Ws2, bs2)


def kernel(x, edge_index, q, W1, b1, W2, b2, Ws1, bs1, Ws2, bs2):
    src = edge_index[0].astype(jnp.int32)
    dst = edge_index[1].astype(jnp.int32)
    pad = EPAD - E
    srcp = jnp.concatenate([src, jnp.zeros((pad,), jnp.int32)])
    dstp = jnp.concatenate([dst, jnp.full((pad,), N, jnp.int32)])
    src_both = jnp.stack([srcp, srcp + N]).reshape(2, 1, EPAD)
    zf = jnp.zeros((ZPT, 128), jnp.float32)
    ones = jnp.ones((BE, 128), jnp.float32)

    s1 = _make_sc_agg(False)(srcp, dstp, x, zf)
    if isinstance(s1, (list, tuple)):
        s1 = s1[0]
    s1 = s1[:, :N]
    cnt = _make_sc_cnt()(dstp, zf, ones)
    if isinstance(cnt, (list, tuple)):
        cnt = cnt[0]
    cnt = cnt[:, :N]

    q2 = q.reshape(1, QD)
    h1, rc = _tc_layer1(x, s1, cnt, q2, W1, b1.reshape(1, H))

    h1f = h1.reshape(2 * N, H // 2)
    s2 = _make_sc_agg(True)(src_both, dstp, h1f, zf)
    if isinstance(s2, (list, tuple)):
        s2 = s2[0]
    s2 = s2[:, :N]

    logits = _tc_layer2(h1, s2, rc, q2, W2, b2.reshape(1, H),
                        Ws1, bs1.reshape(1, 128), Ws2, bs2.reshape(1, 1))
    return logits[:, 0]


# speedup vs baseline: 4.3140x; 1.0978x over previous
import functools

import jax
import jax.numpy as jnp
from jax import lax
from jax.experimental import pallas as pl
from jax.experimental.pallas import tpu as pltpu
from jax.experimental.pallas import tpu_sc as plsc

N = 10000
E = 320000
D = 128
H = 256
QD = 128

NSUB = 16
NCORE = 2
BE = 128
EPAD = 327680
EPS = EPAD // NSUB
NBATCH = EPS // BE
NP = 10112
RPT = 624
ZPT = NP // NSUB

BLK = 2000


KSUB = 2
SUPER = 8
EB = EPAD // BE


@functools.cache
def _make_sc_agg(col_split):
    F = 128
    mesh = plsc.VectorSubcoreMesh(core_axis_name="c", subcore_axis_name="s",
                                  num_cores=NCORE, num_subcores=NSUB)

    def body(src_hbm, dst_hbm, h_hbm, zf_hbm, sum_out,
             src_v, dst_v, rows_v, acc, gsem, ssem):
        c = lax.axis_index("c")
        s = lax.axis_index("s")

        zbase = s * ZPT
        pltpu.sync_copy(zf_hbm, acc.at[pl.ds(zbase, ZPT)])
        plsc.subcore_barrier()

        if col_split:
            rbase0 = s * (EPS // BE)
            nsuper = (EPS // BE) // SUPER
        else:
            wid = s * NCORE + c
            rbase0 = wid * (EPAD // (NSUB * NCORE) // BE)
            nsuper = EPAD // (NSUB * NCORE) // BE // SUPER

        def sup(i, carry):
            ibase = rbase0 + i * SUPER
            eoff = ibase * BE
            if col_split:
                pltpu.sync_copy(src_hbm.at[c, 0, pl.ds(eoff, SUPER * BE)],
                                src_v)
            else:
                pltpu.sync_copy(src_hbm.at[pl.ds(eoff, SUPER * BE)], src_v)
            pltpu.sync_copy(dst_hbm.at[pl.ds(ibase, SUPER)], dst_v)
            for half in range(SUPER // KSUB):
                gh = []
                for j in range(KSUB):
                    sb = half * KSUB + j
                    gh.append(pltpu.async_copy(
                        h_hbm.at[src_v.at[pl.ds(sb * BE, BE)]],
                        rows_v.at[pl.ds(j * BE, BE)], gsem))
                for hd in gh:
                    hd.wait()
                sh = []
                for j in range(KSUB):
                    sb = half * KSUB + j
                    sh.append(pltpu.async_copy(
                        rows_v.at[pl.ds(j * BE, BE)],
                        acc.at[dst_v.at[sb]], ssem, add=True))
                for hd in sh:
                    hd.wait()
            return carry
        lax.fori_loop(0, nsuper, sup, 0)
        plsc.subcore_barrier()

        pltpu.sync_copy(acc.at[pl.ds(zbase, ZPT)],
                        sum_out.at[c, pl.ds(zbase, ZPT)])

    return pl.kernel(
        body,
        out_type=jax.ShapeDtypeStruct((NCORE, NP, F), jnp.float32),
        mesh=mesh,
        scratch_types=[
            pltpu.VMEM((SUPER * BE,), jnp.int32),
            pltpu.VMEM((SUPER, BE), jnp.int32),
            pltpu.VMEM((KSUB * BE, F), jnp.float32),
            pltpu.VMEM_SHARED((NP, F), jnp.float32),
            pltpu.SemaphoreType.DMA,
            pltpu.SemaphoreType.DMA,
        ])


@functools.cache
def _make_sc_cnt():
    mesh = plsc.VectorSubcoreMesh(core_axis_name="c", subcore_axis_name="s",
                                  num_cores=NCORE, num_subcores=NSUB)

    def body(dst_hbm, zf_hbm, ones_hbm, cnt_out, dst_v, ones_v, acc, sem):
        c = lax.axis_index("c")
        s = lax.axis_index("s")
        zbase = s * ZPT
        pltpu.sync_copy(zf_hbm, acc.at[pl.ds(zbase, ZPT)])
        pltpu.sync_copy(ones_hbm, ones_v)
        plsc.subcore_barrier()

        wid = s * NCORE + c
        ebase = wid * (EPAD // (NSUB * NCORE))
        nbatch = EPAD // (NSUB * NCORE) // BE

        def batch(i, carry):
            eoff = ebase + i * BE
            pltpu.sync_copy(dst_hbm.at[pl.ds(eoff, BE)], dst_v)
            pltpu.sync_copy(ones_v, acc.at[dst_v], add=True)
            return carry
        lax.fori_loop(0, nbatch, batch, 0)
        plsc.subcore_barrier()

        pltpu.sync_copy(acc.at[pl.ds(zbase, ZPT)],
                        cnt_out.at[c, pl.ds(zbase, ZPT)])

    return pl.kernel(
        body,
        out_type=jax.ShapeDtypeStruct((NCORE, NP, 128), jnp.float32),
        mesh=mesh,
        scratch_types=[
            pltpu.VMEM((BE,), jnp.int32),
            pltpu.VMEM((BE, 128), jnp.float32),
            pltpu.VMEM_SHARED((NP, 128), jnp.float32),
            pltpu.SemaphoreType.DMA,
        ])


def _l1_body(x_r, s1_r, cnt_r, q_r, W_r, b_r, h1_r, rc_r):
    cnt = cnt_r[0, :, 0:1] + cnt_r[1, :, 0:1] + 1.0
    rc = 1.0 / cnt
    x = x_r[...]
    agg = (s1_r[0] + s1_r[1] + x) * rc
    W = W_r[...]
    acc = jnp.dot(x, W[0:128], preferred_element_type=jnp.float32)
    acc = acc + jnp.dot(agg, W[128:256], preferred_element_type=jnp.float32)
    qc = jnp.dot(q_r[...], W[256:384], preferred_element_type=jnp.float32)
    h = jnp.maximum(acc + qc + b_r[...], 0.0)
    h1_r[0] = h[:, :128]
    h1_r[1] = h[:, 128:]
    rc_r[...] = rc


def _tc_layer1(x, s1, cntT, q2, W1, b1):
    return pl.pallas_call(
        _l1_body,
        grid=(N // BLK,),
        in_specs=[
            pl.BlockSpec((BLK, D), lambda i: (i, 0)),
            pl.BlockSpec((2, BLK, 128), lambda i: (0, i, 0)),
            pl.BlockSpec((2, BLK, 128), lambda i: (0, i, 0)),
            pl.BlockSpec((1, QD), lambda i: (0, 0)),
            pl.BlockSpec((2 * D + QD, H), lambda i: (0, 0)),
            pl.BlockSpec((1, H), lambda i: (0, 0)),
        ],
        out_specs=[
            pl.BlockSpec((2, BLK, H // 2), lambda i: (0, i, 0)),
            pl.BlockSpec((BLK, 1), lambda i: (i, 0)),
        ],
        out_shape=[
            jax.ShapeDtypeStruct((2, N, H // 2), jnp.float32),
            jax.ShapeDtypeStruct((N, 1), jnp.float32),
        ],
    )(x, s1, cntT, q2, W1, b1)


def _l2_body(h1_r, s2_r, rc_r, q_r, W2_r, b2_r, Ws1_r, bs1_r, Ws2_r, bs2_r,
             out_r):
    rc = rc_r[...]
    h1a = h1_r[0]
    h1b = h1_r[1]
    aggA = (s2_r[0] + h1a) * rc
    aggB = (s2_r[1] + h1b) * rc
    W2 = W2_r[...]
    acc = jnp.dot(h1a, W2[0:128], preferred_element_type=jnp.float32)
    acc = acc + jnp.dot(h1b, W2[128:256], preferred_element_type=jnp.float32)
    acc = acc + jnp.dot(aggA, W2[256:384], preferred_element_type=jnp.float32)
    acc = acc + jnp.dot(aggB, W2[384:512], preferred_element_type=jnp.float32)
    qc = jnp.dot(q_r[...], W2[512:640], preferred_element_type=jnp.float32)
    h2 = jnp.maximum(acc + qc + b2_r[...], 0.0)
    Ws1 = Ws1_r[...]
    sacc = jnp.dot(h2, Ws1[0:256], preferred_element_type=jnp.float32)
    sq = jnp.dot(q_r[...], Ws1[256:384], preferred_element_type=jnp.float32)
    sv = jnp.maximum(sacc + sq + bs1_r[...], 0.0)
    out_r[...] = (jnp.dot(sv, Ws2_r[...], preferred_element_type=jnp.float32)
                  + bs2_r[...])


def _tc_layer2(h1, s2, rc, q2, W2, b2, Ws1, bs1, Ws2, bs2):
    return pl.pallas_call(
        _l2_body,
        grid=(N // BLK,),
        in_specs=[
            pl.BlockSpec((2, BLK, H // 2), lambda i: (0, i, 0)),
            pl.BlockSpec((2, BLK, H // 2), lambda i: (0, i, 0)),
            pl.BlockSpec((BLK, 1), lambda i: (i, 0)),
            pl.BlockSpec((1, QD), lambda i: (0, 0)),
            pl.BlockSpec((2 * H + QD, H), lambda i: (0, 0)),
            pl.BlockSpec((1, H), lambda i: (0, 0)),
            pl.BlockSpec((H + QD, 128), lambda i: (0, 0)),
            pl.BlockSpec((1, 128), lambda i: (0, 0)),
            pl.BlockSpec((128, 1), lambda i: (0, 0)),
            pl.BlockSpec((1, 1), lambda i: (0, 0)),
        ],
        out_specs=pl.BlockSpec((BLK, 1), lambda i: (i, 0)),
        out_shape=jax.ShapeDtypeStruct((N, 1), jnp.float32),
    )(h1, s2, rc, q2, W2, b2, Ws1, bs1, Ws2, bs2)


def kernel(x, edge_index, q, W1, b1, W2, b2, Ws1, bs1, Ws2, bs2):
    src = edge_index[0].astype(jnp.int32)
    dst = edge_index[1].astype(jnp.int32)
    pad = EPAD - E
    srcp = jnp.concatenate([src, jnp.zeros((pad,), jnp.int32)])
    dstp = jnp.concatenate([dst, jnp.full((pad,), N, jnp.int32)])
    src_both = jnp.stack([srcp, srcp + N]).reshape(2, 1, EPAD)
    zf = jnp.zeros((ZPT, 128), jnp.float32)
    ones = jnp.ones((BE, 128), jnp.float32)

    dst2d = dstp.reshape(EB, BE)
    s1 = _make_sc_agg(False)(srcp, dst2d, x, zf)
    if isinstance(s1, (list, tuple)):
        s1 = s1[0]
    s1 = s1[:, :N]
    cnt = _make_sc_cnt()(dstp, zf, ones)
    if isinstance(cnt, (list, tuple)):
        cnt = cnt[0]
    cnt = cnt[:, :N]

    q2 = q.reshape(1, QD)
    h1, rc = _tc_layer1(x, s1, cnt, q2, W1, b1.reshape(1, H))

    h1f = h1.reshape(2 * N, H // 2)
    s2 = _make_sc_agg(True)(src_both, dst2d, h1f, zf)
    if isinstance(s2, (list, tuple)):
        s2 = s2[0]
    s2 = s2[:, :N]

    logits = _tc_layer2(h1, s2, rc, q2, W2, b2.reshape(1, H),
                        Ws1, bs1.reshape(1, 128), Ws2, bs2.reshape(1, 1))
    return logits[:, 0]


# speedup vs baseline: 4.5027x; 1.0437x over previous
import functools

import jax
import jax.numpy as jnp
from jax import lax
from jax.experimental import pallas as pl
from jax.experimental.pallas import tpu as pltpu
from jax.experimental.pallas import tpu_sc as plsc

N = 10000
E = 320000
D = 128
H = 256
QD = 128

NSUB = 16
NCORE = 2
BE = 128
EPAD = 327680
EPS = EPAD // NSUB
NBATCH = EPS // BE
NP = 10112
RPT = 624
ZPT = NP // NSUB

BLK = 2000


KSUB = 2
SUPER = 8
EB = EPAD // BE


@functools.cache
def _make_sc_agg(col_split):
    F = 128
    mesh = plsc.VectorSubcoreMesh(core_axis_name="c", subcore_axis_name="s",
                                  num_cores=NCORE, num_subcores=NSUB)

    def body(src_hbm, dst_hbm, h_hbm, zf_hbm, sum_out,
             src_v, dst_v, rows_v, acc, gsem, gsem2, ssem, ssem2):
        c = lax.axis_index("c")
        s = lax.axis_index("s")

        zbase = s * ZPT
        pltpu.sync_copy(zf_hbm, acc.at[pl.ds(zbase, ZPT)])
        plsc.subcore_barrier()

        if col_split:
            rbase0 = s * (EPS // BE)
            nsuper = (EPS // BE) // SUPER
        else:
            wid = s * NCORE + c
            rbase0 = wid * (EPAD // (NSUB * NCORE) // BE)
            nsuper = EPAD // (NSUB * NCORE) // BE // SUPER

        def sup(i, carry):
            ibase = rbase0 + i * SUPER
            eoff = ibase * BE
            if col_split:
                pltpu.sync_copy(src_hbm.at[c, 0, pl.ds(eoff, SUPER * BE)],
                                src_v)
            else:
                pltpu.sync_copy(src_hbm.at[pl.ds(eoff, SUPER * BE)], src_v)
            pltpu.sync_copy(dst_hbm.at[pl.ds(ibase, SUPER)], dst_v)
            gsems = (gsem, gsem2)
            ssems = (ssem, ssem2)
            gh = [None] * SUPER
            sh = [None] * SUPER
            gh[0] = pltpu.async_copy(
                h_hbm.at[src_v.at[pl.ds(0, BE)]],
                rows_v.at[pl.ds(0, BE)], gsems[0])
            for k in range(SUPER):
                r = k % 2
                gh[k].wait()
                sh[k] = pltpu.async_copy(
                    rows_v.at[pl.ds(r * BE, BE)],
                    acc.at[dst_v.at[k]], ssems[r], add=True)
                if k + 1 < SUPER:
                    r2 = (k + 1) % 2
                    if k >= 1:
                        sh[k - 1].wait()
                    gh[k + 1] = pltpu.async_copy(
                        h_hbm.at[src_v.at[pl.ds((k + 1) * BE, BE)]],
                        rows_v.at[pl.ds(r2 * BE, BE)], gsems[r2])
            sh[SUPER - 2].wait()
            sh[SUPER - 1].wait()
            return carry
        lax.fori_loop(0, nsuper, sup, 0)
        plsc.subcore_barrier()

        pltpu.sync_copy(acc.at[pl.ds(zbase, ZPT)],
                        sum_out.at[c, pl.ds(zbase, ZPT)])

    return pl.kernel(
        body,
        out_type=jax.ShapeDtypeStruct((NCORE, NP, F), jnp.float32),
        mesh=mesh,
        scratch_types=[
            pltpu.VMEM((SUPER * BE,), jnp.int32),
            pltpu.VMEM((SUPER, BE), jnp.int32),
            pltpu.VMEM((2 * BE, F), jnp.float32),
            pltpu.VMEM_SHARED((NP, F), jnp.float32),
            pltpu.SemaphoreType.DMA,
            pltpu.SemaphoreType.DMA,
            pltpu.SemaphoreType.DMA,
            pltpu.SemaphoreType.DMA,
        ])


@functools.cache
def _make_sc_cnt():
    mesh = plsc.VectorSubcoreMesh(core_axis_name="c", subcore_axis_name="s",
                                  num_cores=NCORE, num_subcores=NSUB)

    def body(dst_hbm, zf_hbm, ones_hbm, cnt_out, dst_v, ones_v, acc, sem):
        c = lax.axis_index("c")
        s = lax.axis_index("s")
        zbase = s * ZPT
        pltpu.sync_copy(zf_hbm, acc.at[pl.ds(zbase, ZPT)])
        pltpu.sync_copy(ones_hbm, ones_v)
        plsc.subcore_barrier()

        wid = s * NCORE + c
        ebase = wid * (EPAD // (NSUB * NCORE))
        nbatch = EPAD // (NSUB * NCORE) // BE

        def batch(i, carry):
            eoff = ebase + i * BE
            pltpu.sync_copy(dst_hbm.at[pl.ds(eoff, BE)], dst_v)
            pltpu.sync_copy(ones_v, acc.at[dst_v], add=True)
            return carry
        lax.fori_loop(0, nbatch, batch, 0)
        plsc.subcore_barrier()

        pltpu.sync_copy(acc.at[pl.ds(zbase, ZPT)],
                        cnt_out.at[c, pl.ds(zbase, ZPT)])

    return pl.kernel(
        body,
        out_type=jax.ShapeDtypeStruct((NCORE, NP, 128), jnp.float32),
        mesh=mesh,
        scratch_types=[
            pltpu.VMEM((BE,), jnp.int32),
            pltpu.VMEM((BE, 128), jnp.float32),
            pltpu.VMEM_SHARED((NP, 128), jnp.float32),
            pltpu.SemaphoreType.DMA,
        ])


def _l1_body(x_r, s1_r, cnt_r, q_r, W_r, b_r, h1_r, rc_r):
    cnt = cnt_r[0, :, 0:1] + cnt_r[1, :, 0:1] + 1.0
    rc = 1.0 / cnt
    x = x_r[...]
    agg = (s1_r[0] + s1_r[1] + x) * rc
    W = W_r[...]
    acc = jnp.dot(x, W[0:128], preferred_element_type=jnp.float32)
    acc = acc + jnp.dot(agg, W[128:256], preferred_element_type=jnp.float32)
    qc = jnp.dot(q_r[...], W[256:384], preferred_element_type=jnp.float32)
    h = jnp.maximum(acc + qc + b_r[...], 0.0)
    h1_r[0] = h[:, :128]
    h1_r[1] = h[:, 128:]
    rc_r[...] = rc


def _tc_layer1(x, s1, cntT, q2, W1, b1):
    return pl.pallas_call(
        _l1_body,
        grid=(N // BLK,),
        in_specs=[
            pl.BlockSpec((BLK, D), lambda i: (i, 0)),
            pl.BlockSpec((2, BLK, 128), lambda i: (0, i, 0)),
            pl.BlockSpec((2, BLK, 128), lambda i: (0, i, 0)),
            pl.BlockSpec((1, QD), lambda i: (0, 0)),
            pl.BlockSpec((2 * D + QD, H), lambda i: (0, 0)),
            pl.BlockSpec((1, H), lambda i: (0, 0)),
        ],
        out_specs=[
            pl.BlockSpec((2, BLK, H // 2), lambda i: (0, i, 0)),
            pl.BlockSpec((BLK, 1), lambda i: (i, 0)),
        ],
        out_shape=[
            jax.ShapeDtypeStruct((2, N, H // 2), jnp.float32),
            jax.ShapeDtypeStruct((N, 1), jnp.float32),
        ],
    )(x, s1, cntT, q2, W1, b1)


def _l2_body(h1_r, s2_r, rc_r, q_r, W2_r, b2_r, Ws1_r, bs1_r, Ws2_r, bs2_r,
             out_r):
    rc = rc_r[...]
    h1a = h1_r[0]
    h1b = h1_r[1]
    aggA = (s2_r[0] + h1a) * rc
    aggB = (s2_r[1] + h1b) * rc
    W2 = W2_r[...]
    acc = jnp.dot(h1a, W2[0:128], preferred_element_type=jnp.float32)
    acc = acc + jnp.dot(h1b, W2[128:256], preferred_element_type=jnp.float32)
    acc = acc + jnp.dot(aggA, W2[256:384], preferred_element_type=jnp.float32)
    acc = acc + jnp.dot(aggB, W2[384:512], preferred_element_type=jnp.float32)
    qc = jnp.dot(q_r[...], W2[512:640], preferred_element_type=jnp.float32)
    h2 = jnp.maximum(acc + qc + b2_r[...], 0.0)
    Ws1 = Ws1_r[...]
    sacc = jnp.dot(h2, Ws1[0:256], preferred_element_type=jnp.float32)
    sq = jnp.dot(q_r[...], Ws1[256:384], preferred_element_type=jnp.float32)
    sv = jnp.maximum(sacc + sq + bs1_r[...], 0.0)
    out_r[...] = (jnp.dot(sv, Ws2_r[...], preferred_element_type=jnp.float32)
                  + bs2_r[...])


def _tc_layer2(h1, s2, rc, q2, W2, b2, Ws1, bs1, Ws2, bs2):
    return pl.pallas_call(
        _l2_body,
        grid=(N // BLK,),
        in_specs=[
            pl.BlockSpec((2, BLK, H // 2), lambda i: (0, i, 0)),
            pl.BlockSpec((2, BLK, H // 2), lambda i: (0, i, 0)),
            pl.BlockSpec((BLK, 1), lambda i: (i, 0)),
            pl.BlockSpec((1, QD), lambda i: (0, 0)),
            pl.BlockSpec((2 * H + QD, H), lambda i: (0, 0)),
            pl.BlockSpec((1, H), lambda i: (0, 0)),
            pl.BlockSpec((H + QD, 128), lambda i: (0, 0)),
            pl.BlockSpec((1, 128), lambda i: (0, 0)),
            pl.BlockSpec((128, 1), lambda i: (0, 0)),
            pl.BlockSpec((1, 1), lambda i: (0, 0)),
        ],
        out_specs=pl.BlockSpec((BLK, 1), lambda i: (i, 0)),
        out_shape=jax.ShapeDtypeStruct((N, 1), jnp.float32),
    )(h1, s2, rc, q2, W2, b2, Ws1, bs1, Ws2, bs2)


def kernel(x, edge_index, q, W1, b1, W2, b2, Ws1, bs1, Ws2, bs2):
    src = edge_index[0].astype(jnp.int32)
    dst = edge_index[1].astype(jnp.int32)
    pad = EPAD - E
    srcp = jnp.concatenate([src, jnp.zeros((pad,), jnp.int32)])
    dstp = jnp.concatenate([dst, jnp.full((pad,), N, jnp.int32)])
    src_both = jnp.stack([srcp, srcp + N]).reshape(2, 1, EPAD)
    zf = jnp.zeros((ZPT, 128), jnp.float32)
    ones = jnp.ones((BE, 128), jnp.float32)

    dst2d = dstp.reshape(EB, BE)
    s1 = _make_sc_agg(False)(srcp, dst2d, x, zf)
    if isinstance(s1, (list, tuple)):
        s1 = s1[0]
    s1 = s1[:, :N]
    cnt = _make_sc_cnt()(dstp, zf, ones)
    if isinstance(cnt, (list, tuple)):
        cnt = cnt[0]
    cnt = cnt[:, :N]

    q2 = q.reshape(1, QD)
    h1, rc = _tc_layer1(x, s1, cnt, q2, W1, b1.reshape(1, H))

    h1f = h1.reshape(2 * N, H // 2)
    s2 = _make_sc_agg(True)(src_both, dst2d, h1f, zf)
    if isinstance(s2, (list, tuple)):
        s2 = s2[0]
    s2 = s2[:, :N]

    logits = _tc_layer2(h1, s2, rc, q2, W2, b2.reshape(1, H),
                        Ws1, bs1.reshape(1, 128), Ws2, bs2.reshape(1, 1))
    return logits[:, 0]
